# baseline (speedup 1.0000x reference)
import functools

import jax
import jax.numpy as jnp
from jax import lax
from jax.experimental import pallas as pl
from jax.experimental.pallas import tpu as pltpu
from jax.experimental.pallas import tpu_sc as plsc

B = 16384
D = 64
L = 16

_info = plsc.get_sparse_core_info()
NC = _info.num_cores
NS = _info.num_subcores
NW = NC * NS
BPW = B // NW
NCHUNK = BPW // L


def _mf_body(u_id_hbm, i_id_hbm, user_emb_hbm, user_bias_hbm, item_emb_hbm,
             item_bias_hbm, mean_hbm, out_hbm,
             uid_v, iid_v, u_rows, i_rows, bu_v, bi_v, mean_v, out_v, sem):
    wid = lax.axis_index("s") * NC + lax.axis_index("c")
    base = wid * BPW

    pltpu.sync_copy(u_id_hbm.at[pl.ds(base, BPW)], uid_v)
    pltpu.sync_copy(i_id_hbm.at[pl.ds(base, BPW)], iid_v)
    pltpu.sync_copy(mean_hbm, mean_v)
    m = mean_v[...]

    cps = [
        pltpu.async_copy(user_emb_hbm.at[uid_v], u_rows, sem),
        pltpu.async_copy(item_emb_hbm.at[iid_v], i_rows, sem),
        pltpu.async_copy(user_bias_hbm.at[uid_v], bu_v, sem),
        pltpu.async_copy(item_bias_hbm.at[iid_v], bi_v, sem),
    ]
    for cp in cps:
        cp.wait()

    lane = lax.iota(jnp.int32, L)

    def chunk_body(c, _):
        rows = c * L + lane
        acc = plsc.load_gather(u_rows, [rows, lane - lane]) * \
            plsc.load_gather(i_rows, [rows, lane - lane])
        for d in range(1, D):
            col = jnp.full((L,), d, jnp.int32)
            acc = acc + plsc.load_gather(u_rows, [rows, col]) * \
                plsc.load_gather(i_rows, [rows, col])
        off = pl.multiple_of(c * L, L)
        res = acc + bu_v[pl.ds(off, L)] + bi_v[pl.ds(off, L)] + m
        out_v[pl.ds(off, L)] = res
        return 0

    lax.fori_loop(0, NCHUNK, chunk_body, 0)
    pltpu.sync_copy(out_v, out_hbm.at[pl.ds(base, BPW)])


@functools.partial(jax.jit, static_argnames=())
def _mf(u_id, i_id, user_emb, user_bias, item_emb, item_bias, mean):
    mesh = plsc.VectorSubcoreMesh(core_axis_name="c", subcore_axis_name="s")
    kern = functools.partial(
        pl.kernel,
        mesh=mesh,
        compiler_params=pltpu.CompilerParams(
            needs_layout_passes=False, use_tc_tiling_on_sc=False),
        out_type=jax.ShapeDtypeStruct((B,), jnp.float32),
        scratch_types=[
            pltpu.VMEM((BPW,), jnp.int32),
            pltpu.VMEM((BPW,), jnp.int32),
            pltpu.VMEM((BPW, D), jnp.float32),
            pltpu.VMEM((BPW, D), jnp.float32),
            pltpu.VMEM((BPW,), jnp.float32),
            pltpu.VMEM((BPW,), jnp.float32),
            pltpu.VMEM((L,), jnp.float32),
            pltpu.VMEM((BPW,), jnp.float32),
            pltpu.SemaphoreType.DMA,
        ],
    )(_mf_body)
    return kern(u_id, i_id, user_emb, user_bias, item_emb, item_bias, mean)


def kernel(u_id, i_id, user_emb, user_bias, item_emb, item_bias, mean):
    u_id = u_id.astype(jnp.int32)
    i_id = i_id.astype(jnp.int32)
    user_bias = user_bias.reshape(-1)
    item_bias = item_bias.reshape(-1)
    mean_vec = jnp.broadcast_to(mean.reshape(()), (L,))
    return _mf(u_id, i_id, user_emb, user_bias, item_emb, item_bias, mean_vec)


# speedup vs baseline: 1.0247x; 1.0247x over previous
import functools

import jax
import jax.numpy as jnp
from jax import lax
from jax.experimental import pallas as pl
from jax.experimental.pallas import tpu as pltpu
from jax.experimental.pallas import tpu_sc as plsc

B = 16384
D = 64
L = 16

_info = plsc.get_sparse_core_info()
NC = _info.num_cores
NS = _info.num_subcores
NW = NC * NS
BPW = B // NW
NCHUNK = BPW // L

BIAS_ROWS = (1000000 + 64) // L


def _mf_body(u_id_hbm, i_id_hbm, user_emb_hbm, user_bias_hbm, item_emb_hbm,
             item_bias_hbm, mean_hbm, out_hbm,
             uid_v, iid_v, u_rows, i_rows, ur_v, ir_v, bu_rows, bi_rows,
             mean_v, out_v, sem):
    wid = lax.axis_index("s") * NC + lax.axis_index("c")
    base = wid * BPW

    pltpu.sync_copy(u_id_hbm.at[pl.ds(base, BPW)], uid_v)
    pltpu.sync_copy(i_id_hbm.at[pl.ds(base, BPW)], iid_v)
    pltpu.sync_copy(mean_hbm, mean_v)
    m = mean_v[...]

    def bias_idx_body(c, _):
        off = pl.multiple_of(c * L, L)
        ur_v[pl.ds(off, L)] = lax.shift_right_logical(uid_v[pl.ds(off, L)], 4)
        ir_v[pl.ds(off, L)] = lax.shift_right_logical(iid_v[pl.ds(off, L)], 4)
        return 0

    lax.fori_loop(0, NCHUNK, bias_idx_body, 0)

    cps = [
        pltpu.async_copy(user_emb_hbm.at[uid_v], u_rows, sem),
        pltpu.async_copy(item_emb_hbm.at[iid_v], i_rows, sem),
        pltpu.async_copy(user_bias_hbm.at[ur_v], bu_rows, sem),
        pltpu.async_copy(item_bias_hbm.at[ir_v], bi_rows, sem),
    ]
    for cp in cps:
        cp.wait()

    lane = lax.iota(jnp.int32, L)

    def chunk_body(c, _):
        rows = c * L + lane
        acc = plsc.load_gather(u_rows, [rows, lane]) * \
            plsc.load_gather(i_rows, [rows, lane])
        for d in range(1, D):
            col = (lane + d) & (D - 1)
            acc = acc + plsc.load_gather(u_rows, [rows, col]) * \
                plsc.load_gather(i_rows, [rows, col])
        off = pl.multiple_of(c * L, L)
        bu = plsc.load_gather(bu_rows, [rows, uid_v[pl.ds(off, L)] & (L - 1)])
        bi = plsc.load_gather(bi_rows, [rows, iid_v[pl.ds(off, L)] & (L - 1)])
        out_v[pl.ds(off, L)] = acc + bu + bi + m
        return 0

    lax.fori_loop(0, NCHUNK, chunk_body, 0)
    pltpu.sync_copy(out_v, out_hbm.at[pl.ds(base, BPW)])


@jax.jit
def _mf(u_id, i_id, user_emb, user_bias, item_emb, item_bias, mean):
    u_id = u_id.astype(jnp.int32)
    i_id = i_id.astype(jnp.int32)
    user_bias = jnp.pad(user_bias, ((0, 64), (0, 0))).reshape(BIAS_ROWS, L)
    item_bias = jnp.pad(item_bias, ((0, 64), (0, 0))).reshape(BIAS_ROWS, L)
    mean = jnp.broadcast_to(mean.reshape(()), (L,))
    mesh = plsc.VectorSubcoreMesh(core_axis_name="c", subcore_axis_name="s")
    kern = functools.partial(
        pl.kernel,
        mesh=mesh,
        compiler_params=pltpu.CompilerParams(
            needs_layout_passes=False, use_tc_tiling_on_sc=False),
        out_type=jax.ShapeDtypeStruct((B,), jnp.float32),
        scratch_types=[
            pltpu.VMEM((BPW,), jnp.int32),
            pltpu.VMEM((BPW,), jnp.int32),
            pltpu.VMEM((BPW, D), jnp.float32),
            pltpu.VMEM((BPW, D), jnp.float32),
            pltpu.VMEM((BPW,), jnp.int32),
            pltpu.VMEM((BPW,), jnp.int32),
            pltpu.VMEM((BPW, L), jnp.float32),
            pltpu.VMEM((BPW, L), jnp.float32),
            pltpu.VMEM((L,), jnp.float32),
            pltpu.VMEM((BPW,), jnp.float32),
            pltpu.SemaphoreType.DMA,
        ],
    )(_mf_body)
    return kern(u_id, i_id, user_emb, user_bias, item_emb, item_bias, mean)


def kernel(u_id, i_id, user_emb, user_bias, item_emb, item_bias, mean):
    return _mf(u_id, i_id, user_emb, user_bias, item_emb, item_bias, mean)


# speedup vs baseline: 1.4180x; 1.3838x over previous
import functools

import jax
import jax.numpy as jnp
from jax import lax
from jax.experimental import pallas as pl
from jax.experimental.pallas import tpu as pltpu
from jax.experimental.pallas import tpu_sc as plsc

B = 16384
D = 64
L = 16

_info = plsc.get_sparse_core_info()
NC = _info.num_cores
NS = _info.num_subcores
NW = NC * NS
BPW = B // NW
NCHUNK = BPW // L

BIAS_COLS = 128
BIAS_ROWS = (1000000 + BIAS_COLS * 8) // BIAS_COLS
BIAS_PAD = BIAS_ROWS * BIAS_COLS - 1000000


def _mf_body(u_id_hbm, i_id_hbm, user_emb_hbm, user_bias_hbm, item_emb_hbm,
             item_bias_hbm, mean_hbm, out_hbm,
             uid_v, iid_v, u_rows, i_rows, bu_rows, bi_rows, bu_c, bi_c,
             mean_v, out_v, ids_sh, uid_s, iid_s, sem, bsem):
    sid = lax.axis_index("s")
    wid = sid * NC + lax.axis_index("c")
    base = wid * BPW

    pltpu.sync_copy(u_id_hbm.at[pl.ds(base, BPW)], uid_v)
    pltpu.sync_copy(i_id_hbm.at[pl.ds(base, BPW)], iid_v)
    pltpu.sync_copy(uid_v, ids_sh.at[sid, 0])
    pltpu.sync_copy(iid_v, ids_sh.at[sid, 1])
    pltpu.sync_copy(ids_sh.at[sid, 0], uid_s)
    pltpu.sync_copy(ids_sh.at[sid, 1], iid_s)
    pltpu.sync_copy(mean_hbm, mean_v)
    m = mean_v[...]

    lane = lax.iota(jnp.int32, L)
    BPH = BPW // 2
    CHK = 128

    for h in range(2):
        hb = h * BPH

        def fire_body(k, _, hb=hb):
            pltpu.make_async_copy(
                user_emb_hbm.at[pl.ds(uid_s[hb + k], 1)],
                u_rows.at[pl.ds(k, 1)], sem).start()
            pltpu.make_async_copy(
                item_emb_hbm.at[pl.ds(iid_s[hb + k], 1)],
                i_rows.at[pl.ds(k, 1)], sem).start()
            return 0

        lax.fori_loop(0, BPH, fire_body, 0)

        for r in range(BPH // CHK):
            def bias_fire(k, _, hb=hb, r=r):
                pltpu.make_async_copy(
                    user_bias_hbm.at[pl.ds(uid_s[hb + r * CHK + k] >> 7, 1)],
                    bu_rows.at[pl.ds(k, 1)], bsem).start()
                pltpu.make_async_copy(
                    item_bias_hbm.at[pl.ds(iid_s[hb + r * CHK + k] >> 7, 1)],
                    bi_rows.at[pl.ds(k, 1)], bsem).start()
                return 0

            lax.fori_loop(0, CHK, bias_fire, 0)

            def bias_drain(k, _):
                pltpu.make_async_copy(
                    user_bias_hbm.at[pl.ds(0, 1)], bu_rows.at[pl.ds(k, 1)],
                    bsem).wait()
                pltpu.make_async_copy(
                    item_bias_hbm.at[pl.ds(0, 1)], bi_rows.at[pl.ds(k, 1)],
                    bsem).wait()
                return 0

            lax.fori_loop(0, CHK, bias_drain, 0)

            def bias_extract(c, _, hb=hb, r=r):
                loc = c * L + lane
                off = pl.multiple_of(hb + r * CHK + c * L, L)
                bu_c[pl.ds(off, L)] = plsc.load_gather(
                    bu_rows, [loc, uid_v[pl.ds(off, L)] & (BIAS_COLS - 1)])
                bi_c[pl.ds(off, L)] = plsc.load_gather(
                    bi_rows, [loc, iid_v[pl.ds(off, L)] & (BIAS_COLS - 1)])
                return 0

            lax.fori_loop(0, CHK // L, bias_extract, 0)

        def drain_body(k, _):
            pltpu.make_async_copy(
                user_emb_hbm.at[pl.ds(0, 1)], u_rows.at[pl.ds(k, 1)], sem
            ).wait()
            pltpu.make_async_copy(
                item_emb_hbm.at[pl.ds(0, 1)], i_rows.at[pl.ds(k, 1)], sem
            ).wait()
            return 0

        lax.fori_loop(0, BPH, drain_body, 0)

        def chunk_body(c, _, hb=hb):
            rows = c * L + lane
            off = pl.multiple_of(hb + c * L, L)
            acc = plsc.load_gather(u_rows, [rows, lane]) * \
                plsc.load_gather(i_rows, [rows, lane])
            for d in range(1, D):
                col = (lane + d) & (D - 1)
                acc = acc + plsc.load_gather(u_rows, [rows, col]) * \
                    plsc.load_gather(i_rows, [rows, col])
            out_v[pl.ds(off, L)] = acc + bu_c[pl.ds(off, L)] + \
                bi_c[pl.ds(off, L)] + m
            return 0

        lax.fori_loop(0, BPH // L, chunk_body, 0)

    pltpu.sync_copy(out_v, out_hbm.at[pl.ds(base, BPW)])


@jax.jit
def _mf(u_id, i_id, user_emb, user_bias, item_emb, item_bias, mean):
    u_id = u_id.astype(jnp.int32)
    i_id = i_id.astype(jnp.int32)
    user_bias = jnp.pad(user_bias, ((0, BIAS_PAD), (0, 0))).reshape(
        BIAS_ROWS, BIAS_COLS)
    item_bias = jnp.pad(item_bias, ((0, BIAS_PAD), (0, 0))).reshape(
        BIAS_ROWS, BIAS_COLS)
    mean = jnp.broadcast_to(mean.reshape(()), (L,))
    mesh = plsc.VectorSubcoreMesh(core_axis_name="c", subcore_axis_name="s")
    kern = functools.partial(
        pl.kernel,
        mesh=mesh,
        compiler_params=pltpu.CompilerParams(
            needs_layout_passes=False, use_tc_tiling_on_sc=True),
        out_type=jax.ShapeDtypeStruct((B,), jnp.float32),
        scratch_types=[
            pltpu.VMEM((BPW,), jnp.int32),
            pltpu.VMEM((BPW,), jnp.int32),
            pltpu.VMEM((BPW // 2, D), jnp.float32),
            pltpu.VMEM((BPW // 2, D), jnp.float32),
            pltpu.VMEM((128, BIAS_COLS), jnp.float32),
            pltpu.VMEM((128, BIAS_COLS), jnp.float32),
            pltpu.VMEM((BPW,), jnp.float32),
            pltpu.VMEM((BPW,), jnp.float32),
            pltpu.VMEM((L,), jnp.float32),
            pltpu.VMEM((BPW,), jnp.float32),
            pltpu.VMEM_SHARED((NS, 2, BPW), jnp.int32),
            pltpu.SMEM((BPW,), jnp.int32),
            pltpu.SMEM((BPW,), jnp.int32),
            pltpu.SemaphoreType.DMA,
            pltpu.SemaphoreType.DMA,
        ],
    )(_mf_body)
    return kern(u_id, i_id, user_emb, user_bias, item_emb, item_bias, mean)


def kernel(u_id, i_id, user_emb, user_bias, item_emb, item_bias, mean):
    return _mf(u_id, i_id, user_emb, user_bias, item_emb, item_bias, mean)
